# baseline (speedup 1.0000x reference)
import functools

import jax
import jax.numpy as jnp
from jax.experimental import pallas as pl
from jax.experimental.pallas import tpu as pltpu

_LANES = 128


def _router_body(x_ref, wg_ref, comb_ref, aux_ref):
    x = x_ref[...]
    wg = wg_ref[...]
    logits = jnp.dot(x, wg, preferred_element_type=jnp.float32)
    S = x.shape[0]
    lane = jax.lax.broadcasted_iota(jnp.int32, (S, _LANES), 1)
    E = 8
    neg = jnp.full_like(logits, -jnp.inf)
    logits = jnp.where(lane < E, logits, neg)
    m = jnp.max(logits, axis=1, keepdims=True)
    ex = jnp.exp(logits - m)
    probs = ex / jnp.sum(ex, axis=1, keepdims=True)

    m1 = jnp.max(probs, axis=1, keepdims=True)
    big = jnp.int32(10 ** 9)
    i1 = jnp.min(jnp.where(probs == m1, lane, big), axis=1, keepdims=True)
    probs_m = jnp.where(lane == i1, -1.0, probs)
    m2 = jnp.max(probs_m, axis=1, keepdims=True)
    i2 = jnp.min(jnp.where(probs_m == m2, lane, big), axis=1, keepdims=True)

    denom = m1 + m2
    w1 = m1 / denom
    w2 = m2 / denom
    oh1 = (lane == i1).astype(jnp.float32)
    oh2 = (lane == i2).astype(jnp.float32)
    comb = w1 * oh1 + w2 * oh2
    comb_ref[...] = comb

    counts = jnp.sum(oh1 + oh2, axis=0, keepdims=True)
    avg_prob = jnp.mean(probs, axis=0, keepdims=True)
    aux_ref[0, 0] = jnp.float32(E) * jnp.sum(counts * avg_prob)


def _ffn_body(comb_ref, x_ref, w1_ref, b1_ref, w2_ref, b2_ref, out_ref,
              acc_ref, *, n_dt):
    e = pl.program_id(0)
    dt = pl.program_id(1)
    st = pl.program_id(2)
    ts = x_ref.shape[0]
    x = x_ref[...]
    w1 = w1_ref[0]
    b1 = b1_ref[0]
    w2 = w2_ref[0]
    b2 = b2_ref[0]

    h = jnp.dot(x, w1, preferred_element_type=jnp.float32) + b1
    h = h * 0.5 * (1.0 + jax.lax.erf(h * 0.7071067811865476))
    part = jnp.dot(h.astype(jnp.bfloat16), w2,
                   preferred_element_type=jnp.float32)
    part = jnp.where(dt == 0, part + b2, part)

    lane = jax.lax.broadcasted_iota(jnp.int32, (1, _LANES), 1)
    oh_e = (lane == e).astype(jnp.float32)
    c = jnp.sum(comb_ref[...] * oh_e, axis=1, keepdims=True)
    contrib = c * part

    first = jnp.logical_and(e == 0, dt == 0)
    rows = pl.ds(st * ts, ts)

    @pl.when(first)
    def _():
        acc_ref[rows, :] = contrib

    @pl.when(jnp.logical_not(first))
    def _():
        acc_ref[rows, :] = acc_ref[rows, :] + contrib

    out_ref[...] = acc_ref[rows, :]


def kernel(x, Wg, W1, b1, W2, b2):
    B, S, H = x.shape
    E, _, D = W1.shape
    x2 = x.reshape(S, H)

    wg_pad = jnp.zeros((H, _LANES), jnp.float32).at[:, :E].set(Wg)

    comb, aux = pl.pallas_call(
        _router_body,
        out_shape=(
            jax.ShapeDtypeStruct((S, _LANES), jnp.float32),
            jax.ShapeDtypeStruct((1, 1), jnp.float32),
        ),
        in_specs=[
            pl.BlockSpec(memory_space=pltpu.VMEM),
            pl.BlockSpec(memory_space=pltpu.VMEM),
        ],
        out_specs=(
            pl.BlockSpec(memory_space=pltpu.VMEM),
            pl.BlockSpec(memory_space=pltpu.SMEM),
        ),
    )(x2, wg_pad)

    TS = 128
    DB = 1024
    n_st = S // TS
    n_dt = D // DB

    out = pl.pallas_call(
        functools.partial(_ffn_body, n_dt=n_dt),
        grid=(E, n_dt, n_st),
        in_specs=[
            pl.BlockSpec((TS, _LANES), lambda e, dt, st: (st, 0)),
            pl.BlockSpec((TS, H), lambda e, dt, st: (st, 0)),
            pl.BlockSpec((1, H, DB), lambda e, dt, st: (e, 0, dt)),
            pl.BlockSpec((1, 1, DB), lambda e, dt, st: (e, 0, dt)),
            pl.BlockSpec((1, DB, H), lambda e, dt, st: (e, dt, 0)),
            pl.BlockSpec((1, 1, H), lambda e, dt, st: (e, 0, 0)),
        ],
        out_specs=pl.BlockSpec((TS, H), lambda e, dt, st: (st, 0)),
        out_shape=jax.ShapeDtypeStruct((S, H), jnp.float32),
        scratch_shapes=[pltpu.VMEM((S, H), jnp.float32)],
    )(comb, x2.astype(jnp.bfloat16), W1.astype(jnp.bfloat16),
      b1.reshape(E, 1, D), W2.astype(jnp.bfloat16), b2.reshape(E, 1, H))

    return out.reshape(B, S, H), aux[0, 0]


# speedup vs baseline: 2.0762x; 2.0762x over previous
import jax
import jax.numpy as jnp
from jax import lax
from jax.experimental import pallas as pl
from jax.experimental.pallas import tpu as pltpu
from jax.experimental.pallas import tpu_sc as plsc

_LANES = 128
_S = 2048
_H = 1024
_D = 2048
_E = 8
_P = 2 * _S
_TS = 128
_NT = 40
_NP = _NT * _TS
_NSLICE = 32


def _router_body(x_ref, wg_ref, eo_ref, wo_ref, aux_ref):
    x = x_ref[...]
    wg = wg_ref[...]
    logits = jnp.dot(x, wg, preferred_element_type=jnp.float32)
    S = x.shape[0]
    lane = jax.lax.broadcasted_iota(jnp.int32, (S, _LANES), 1)
    neg = jnp.full_like(logits, -jnp.inf)
    logits = jnp.where(lane < _E, logits, neg)
    m = jnp.max(logits, axis=1, keepdims=True)
    ex = jnp.exp(logits - m)
    probs = ex / jnp.sum(ex, axis=1, keepdims=True)

    m1 = jnp.max(probs, axis=1, keepdims=True)
    big = jnp.int32(10 ** 9)
    i1 = jnp.min(jnp.where(probs == m1, lane, big), axis=1, keepdims=True)
    probs_m = jnp.where(lane == i1, -1.0, probs)
    m2 = jnp.max(probs_m, axis=1, keepdims=True)
    i2 = jnp.min(jnp.where(probs_m == m2, lane, big), axis=1, keepdims=True)

    denom = m1 + m2
    w1 = m1 / denom
    w2 = m2 / denom

    zi = jnp.zeros_like(lane)
    eo_ref[...] = jnp.where(lane == 0, i1, jnp.where(lane == 1, i2, zi))
    zf = jnp.zeros_like(probs)
    wo_ref[...] = jnp.where(lane == 0, w1, jnp.where(lane == 1, w2, zf))

    oh1 = (lane == i1).astype(jnp.float32)
    oh2 = (lane == i2).astype(jnp.float32)
    counts = jnp.sum(oh1 + oh2, axis=0, keepdims=True)
    avg_prob = jnp.mean(probs, axis=0, keepdims=True)
    aux_ref[0, 0] = jnp.float32(_E) * jnp.sum(counts * avg_prob)


def _lane16():
    return jax.lax.broadcasted_iota(jnp.int32, (16,), 0)


def _splat_last(s):
    return plsc.cummax(lax.rev(s, (0,)))


def _splat_lane_i(xvec, e):
    t = jnp.where(_lane16() == e, xvec, jnp.int32(-2147483648))
    return _splat_last(plsc.cummax(t))


def _splat_lane_f(xvec, e):
    t = jnp.where(_lane16() == e, xvec, -jnp.inf)
    return _splat_last(plsc.cummax(t))


def _count_slice(eid_v, base):
    lane = _lane16()
    cvec = jnp.zeros((16,), jnp.int32)
    for v in range(8):
        ev = eid_v[pl.ds(base + v * 16, 16)]
        for e in range(_E):
            mi = (ev == e).astype(jnp.int32)
            tot = _splat_last(jnp.cumsum(mi))
            cvec = jnp.where(lane == e, cvec + tot, cvec)
    return cvec


def _dispatch_body(eids_hbm, x_hbm, pos_hbm, texp_hbm, xs_hbm,
                   eid_v, cnt2_v, counts_v, prefix_v, beforeq_v, totincl_v,
                   pos2_v, texp_v, xrow_v, shared, sem):
    cid = lax.axis_index("c")
    sid = lax.axis_index("s")
    lane = _lane16()

    pltpu.sync_copy(eids_hbm.at[pl.ds(sid * 256, 256)], eid_v)
    for half in range(2):
        cnt2_v[pl.ds(half * 16, 16)] = _count_slice(eid_v, half * 128)
        pltpu.sync_copy(cnt2_v.at[pl.ds(half * 16, 16)],
                        shared.at[pl.ds((2 * sid + half) * 16, 16)])
    plsc.subcore_barrier()

    @pl.when(sid == 0)
    def _():
        pltpu.sync_copy(shared.at[pl.ds(0, 512)], counts_v)
        acc = jnp.zeros((16,), jnp.int32)
        for r in range(_NSLICE):
            prefix_v[pl.ds(r * 16, 16)] = acc
            acc = acc + counts_v[pl.ds(r * 16, 16)]
        prefix_v[pl.ds(512, 16)] = acc
        pltpu.sync_copy(prefix_v, shared.at[pl.ds(512, 528)])
    plsc.subcore_barrier()

    q = 2 * sid + cid
    pltpu.sync_copy(shared.at[pl.ds(512 + q * 16, 16)], beforeq_v)
    pltpu.sync_copy(shared.at[pl.ds(1024, 16)], totincl_v)
    before = beforeq_v[...]
    total = totincl_v[...]
    padded = jnp.bitwise_and(total + 127, jnp.int32(-128))
    incl = jnp.cumsum(padded)
    base = incl - padded
    start = base + before

    @pl.when(q == 0)
    def _():
        for j in range(3):
            tv = (lane + j * 16) * 128
            cnt = jnp.zeros((16,), jnp.int32)
            for e in range(_E):
                ie = _splat_lane_i(incl, e)
                cnt = cnt + (tv >= ie).astype(jnp.int32)
            texp_v[pl.ds(j * 16, 16)] = jnp.minimum(cnt, _E - 1)
        pltpu.sync_copy(texp_v, texp_hbm)

    pltpu.sync_copy(eids_hbm.at[pl.ds(q * 128, 128)],
                    eid_v.at[pl.ds(0, 128)])
    run = start
    for v in range(8):
        ev = eid_v[pl.ds(v * 16, 16)]
        posv = jnp.zeros((16,), jnp.int32)
        for e in range(_E):
            mv = ev == e
            s = jnp.cumsum(mv.astype(jnp.int32))
            re = _splat_lane_i(run, e)
            posv = jnp.where(mv, re + s - 1, posv)
            run = jnp.where(lane == e, run + _splat_last(s), run)
        pos2_v[v // 2, pl.ds((v % 2) * 16, 16)] = posv

    for ch in range(4):
        pltpu.sync_copy(pos2_v.at[ch],
                        pos_hbm.at[pl.ds(q * 128 + ch * 32, 32)])

    t0 = lax.rem(q, 16) * 128
    for ch in range(4):
        pltpu.sync_copy(x_hbm.at[pl.ds(t0 + ch * 32, 32), :], xrow_v)
        pltpu.async_copy(xrow_v, xs_hbm.at[pos2_v.at[ch]], sem).wait()


def _ffn_body(texp_ref, xs_ref, w1_ref, b1_ref, w2_ref, b2_ref, ys_ref):
    xv = xs_ref[...]
    w1 = w1_ref[0]
    b1 = b1_ref[0]
    w2 = w2_ref[0]
    b2 = b2_ref[0]
    h = jnp.dot(xv, w1, preferred_element_type=jnp.float32) + b1
    h = h * 0.5 * (1.0 + jax.lax.erf(h * 0.7071067811865476))
    ys_ref[...] = jnp.dot(h, w2, preferred_element_type=jnp.float32) + b2


def _combine_body(ys_hbm, pos_hbm, wn_hbm, out_hbm,
                  idx0_v, idx1_v, w0_v, w1_v, r0_v, r1_v, o_v, sem):
    cid = lax.axis_index("c")
    sid = lax.axis_index("s")
    wid = 2 * sid + cid
    tok0 = wid * 64
    pltpu.sync_copy(pos_hbm.at[pl.ds(tok0, 64)], idx0_v)
    pltpu.sync_copy(pos_hbm.at[pl.ds(_S + tok0, 64)], idx1_v)
    pltpu.sync_copy(wn_hbm.at[pl.ds(tok0, 64)], w0_v)
    pltpu.sync_copy(wn_hbm.at[pl.ds(_S + tok0, 64)], w1_v)
    for ch in range(2):
        pltpu.async_copy(ys_hbm.at[idx0_v.at[pl.ds(ch * 32, 32)]],
                         r0_v, sem).wait()
        pltpu.async_copy(ys_hbm.at[idx1_v.at[pl.ds(ch * 32, 32)]],
                         r1_v, sem).wait()
        for i in range(32):
            w0g = w0_v[pl.ds(ch * 32 + (i // 16) * 16, 16)]
            w1g = w1_v[pl.ds(ch * 32 + (i // 16) * 16, 16)]
            w0s = _splat_lane_f(w0g, i % 16)
            w1s = _splat_lane_f(w1g, i % 16)

            def kb(kk, _, i=i, w0s=w0s, w1s=w1s):
                off = pl.multiple_of(kk * 64, 64)
                for u in range(4):
                    a = r0_v[i, pl.ds(off + u * 16, 16)]
                    b = r1_v[i, pl.ds(off + u * 16, 16)]
                    o_v[i, pl.ds(off + u * 16, 16)] = a * w0s + b * w1s
                return 0

            lax.fori_loop(0, _H // 64, kb, 0)
        pltpu.sync_copy(o_v, out_hbm.at[pl.ds(tok0 + ch * 32, 32), :])


def kernel(x, Wg, W1, b1, W2, b2):
    B, S, H = x.shape
    E, _, D = W1.shape
    x2 = x.reshape(S, H)

    wg_pad = jnp.zeros((H, _LANES), jnp.float32).at[:, :E].set(Wg)

    eo, wo, aux = pl.pallas_call(
        _router_body,
        out_shape=(
            jax.ShapeDtypeStruct((S, _LANES), jnp.int32),
            jax.ShapeDtypeStruct((S, _LANES), jnp.float32),
            jax.ShapeDtypeStruct((1, 1), jnp.float32),
        ),
        in_specs=[
            pl.BlockSpec(memory_space=pltpu.VMEM),
            pl.BlockSpec(memory_space=pltpu.VMEM),
        ],
        out_specs=(
            pl.BlockSpec(memory_space=pltpu.VMEM),
            pl.BlockSpec(memory_space=pltpu.VMEM),
            pl.BlockSpec(memory_space=pltpu.SMEM),
        ),
    )(x2, wg_pad)

    eflat = jnp.concatenate([eo[:, 0], eo[:, 1]], axis=0)
    wflat = jnp.concatenate([wo[:, 0], wo[:, 1]], axis=0)

    mesh = plsc.VectorSubcoreMesh(core_axis_name="c", subcore_axis_name="s",
                                  num_cores=2, num_subcores=16)

    dispatch = pl.kernel(
        _dispatch_body,
        out_type=(
            jax.ShapeDtypeStruct((_P,), jnp.int32),
            jax.ShapeDtypeStruct((48,), jnp.int32),
            jax.ShapeDtypeStruct((_NP, H), jnp.float32),
        ),
        mesh=mesh,
        scratch_types=(
            pltpu.VMEM((256,), jnp.int32),
            pltpu.VMEM((32,), jnp.int32),
            pltpu.VMEM((512,), jnp.int32),
            pltpu.VMEM((528,), jnp.int32),
            pltpu.VMEM((16,), jnp.int32),
            pltpu.VMEM((16,), jnp.int32),
            pltpu.VMEM((4, 32), jnp.int32),
            pltpu.VMEM((48,), jnp.int32),
            pltpu.VMEM((32, H), jnp.float32),
            pltpu.VMEM_SHARED((1040,), jnp.int32),
            pltpu.SemaphoreType.DMA,
        ),
        compiler_params=pltpu.CompilerParams(needs_layout_passes=False),
    )
    pos, texp, xs = dispatch(eflat, x2)

    grid_spec = pltpu.PrefetchScalarGridSpec(
        num_scalar_prefetch=1,
        grid=(_NT,),
        in_specs=[
            pl.BlockSpec((_TS, H), lambda t, te: (t, 0)),
            pl.BlockSpec((1, H, D), lambda t, te: (te[t], 0, 0)),
            pl.BlockSpec((1, 1, D), lambda t, te: (te[t], 0, 0)),
            pl.BlockSpec((1, D, H), lambda t, te: (te[t], 0, 0)),
            pl.BlockSpec((1, 1, H), lambda t, te: (te[t], 0, 0)),
        ],
        out_specs=pl.BlockSpec((_TS, H), lambda t, te: (t, 0)),
    )
    ys = pl.pallas_call(
        _ffn_body,
        grid_spec=grid_spec,
        out_shape=jax.ShapeDtypeStruct((_NP, H), jnp.float32),
    )(texp, xs, W1, b1.reshape(E, 1, D), W2, b2.reshape(E, 1, H))

    combine = pl.kernel(
        _combine_body,
        out_type=jax.ShapeDtypeStruct((S, H), jnp.float32),
        mesh=mesh,
        scratch_types=(
            pltpu.VMEM((64,), jnp.int32),
            pltpu.VMEM((64,), jnp.int32),
            pltpu.VMEM((64,), jnp.float32),
            pltpu.VMEM((64,), jnp.float32),
            pltpu.VMEM((32, H), jnp.float32),
            pltpu.VMEM((32, H), jnp.float32),
            pltpu.VMEM((32, H), jnp.float32),
            pltpu.SemaphoreType.DMA,
        ),
        compiler_params=pltpu.CompilerParams(needs_layout_passes=False),
    )
    out = combine(ys, pos, wflat)

    return out.reshape(B, S, H), aux[0, 0]


# speedup vs baseline: 2.1214x; 1.0218x over previous
import jax
import jax.numpy as jnp
from jax import lax
from jax.experimental import pallas as pl
from jax.experimental.pallas import tpu as pltpu
from jax.experimental.pallas import tpu_sc as plsc

_LANES = 128
_S = 2048
_H = 1024
_D = 2048
_E = 8
_P = 2 * _S
_TS = 128
_NT = 40
_NP = _NT * _TS
_NSLICE = 32


def _router_body(x_ref, wg_ref, eo_ref, wo_ref, aux_ref):
    x = x_ref[...]
    wg = wg_ref[...]
    logits = jnp.dot(x, wg, preferred_element_type=jnp.float32)
    S = x.shape[0]
    lane = jax.lax.broadcasted_iota(jnp.int32, (S, _LANES), 1)
    neg = jnp.full_like(logits, -jnp.inf)
    logits = jnp.where(lane < _E, logits, neg)
    m = jnp.max(logits, axis=1, keepdims=True)
    ex = jnp.exp(logits - m)
    probs = ex / jnp.sum(ex, axis=1, keepdims=True)

    m1 = jnp.max(probs, axis=1, keepdims=True)
    big = jnp.int32(10 ** 9)
    i1 = jnp.min(jnp.where(probs == m1, lane, big), axis=1, keepdims=True)
    probs_m = jnp.where(lane == i1, -1.0, probs)
    m2 = jnp.max(probs_m, axis=1, keepdims=True)
    i2 = jnp.min(jnp.where(probs_m == m2, lane, big), axis=1, keepdims=True)

    denom = m1 + m2
    w1 = m1 / denom
    w2 = m2 / denom

    zi = jnp.zeros_like(lane)
    eo_ref[...] = jnp.where(lane == 0, i1, jnp.where(lane == 1, i2, zi))
    zf = jnp.zeros_like(probs)
    wo_ref[...] = jnp.where(lane == 0, w1, jnp.where(lane == 1, w2, zf))

    oh1 = (lane == i1).astype(jnp.float32)
    oh2 = (lane == i2).astype(jnp.float32)
    counts = jnp.sum(oh1 + oh2, axis=0, keepdims=True)
    avg_prob = jnp.mean(probs, axis=0, keepdims=True)
    aux_ref[0, 0] = jnp.float32(_E) * jnp.sum(counts * avg_prob)


def _lane16():
    return jax.lax.broadcasted_iota(jnp.int32, (16,), 0)


def _splat_last(s):
    return plsc.cummax(lax.rev(s, (0,)))


def _splat_lane_i(xvec, e):
    t = jnp.where(_lane16() == e, xvec, jnp.int32(-2147483648))
    return _splat_last(plsc.cummax(t))


def _splat_lane_f(xvec, e):
    t = jnp.where(_lane16() == e, xvec, -jnp.inf)
    return _splat_last(plsc.cummax(t))


def _count_slice(eid_v, base):
    lane = _lane16()
    cvec = jnp.zeros((16,), jnp.int32)
    for v in range(8):
        ev = eid_v[pl.ds(base + v * 16, 16)]
        for e in range(_E):
            mi = (ev == e).astype(jnp.int32)
            tot = _splat_last(jnp.cumsum(mi))
            cvec = jnp.where(lane == e, cvec + tot, cvec)
    return cvec


def _dispatch_body(eids_hbm, x_hbm, pos_hbm, texp_hbm, xs_hbm,
                   eid_v, cnt2_v, counts_v, prefix_v, beforeq_v, totincl_v,
                   pos2_v, texp_v, xrow_v, shared, sem):
    cid = lax.axis_index("c")
    sid = lax.axis_index("s")
    lane = _lane16()

    pltpu.sync_copy(eids_hbm.at[pl.ds(sid * 256, 256)], eid_v)
    for half in range(2):
        cnt2_v[pl.ds(half * 16, 16)] = _count_slice(eid_v, half * 128)
        pltpu.sync_copy(cnt2_v.at[pl.ds(half * 16, 16)],
                        shared.at[pl.ds((2 * sid + half) * 16, 16)])
    plsc.subcore_barrier()

    @pl.when(sid == 0)
    def _():
        pltpu.sync_copy(shared.at[pl.ds(0, 512)], counts_v)
        acc = jnp.zeros((16,), jnp.int32)
        for r in range(_NSLICE):
            prefix_v[pl.ds(r * 16, 16)] = acc
            acc = acc + counts_v[pl.ds(r * 16, 16)]
        prefix_v[pl.ds(512, 16)] = acc
        pltpu.sync_copy(prefix_v, shared.at[pl.ds(512, 528)])
    plsc.subcore_barrier()

    q = 2 * sid + cid
    pltpu.sync_copy(shared.at[pl.ds(512 + q * 16, 16)], beforeq_v)
    pltpu.sync_copy(shared.at[pl.ds(1024, 16)], totincl_v)
    before = beforeq_v[...]
    total = totincl_v[...]
    padded = jnp.bitwise_and(total + 127, jnp.int32(-128))
    incl = jnp.cumsum(padded)
    base = incl - padded
    start = base + before

    @pl.when(q == 0)
    def _():
        top = _splat_lane_i(incl, _E - 1)
        for j in range(3):
            tv = (lane + j * 16) * 128
            cnt = jnp.zeros((16,), jnp.int32)
            for e in range(_E):
                ie = _splat_lane_i(incl, e)
                cnt = cnt + (tv >= ie).astype(jnp.int32)
            texp_v[pl.ds(j * 16, 16)] = jnp.minimum(cnt, _E - 1)
            texp_v[pl.ds(48 + j * 16, 16)] = (tv < top).astype(jnp.int32)
        pltpu.sync_copy(texp_v, texp_hbm)

    pltpu.sync_copy(eids_hbm.at[pl.ds(q * 128, 128)],
                    eid_v.at[pl.ds(0, 128)])
    run = start
    for v in range(8):
        ev = eid_v[pl.ds(v * 16, 16)]
        posv = jnp.zeros((16,), jnp.int32)
        for e in range(_E):
            mv = ev == e
            s = jnp.cumsum(mv.astype(jnp.int32))
            re = _splat_lane_i(run, e)
            posv = jnp.where(mv, re + s - 1, posv)
            run = jnp.where(lane == e, run + _splat_last(s), run)
        pos2_v[v // 2, pl.ds((v % 2) * 16, 16)] = posv

    for ch in range(4):
        pltpu.sync_copy(pos2_v.at[ch],
                        pos_hbm.at[pl.ds(q * 128 + ch * 32, 32)])

    t0 = lax.rem(q, 16) * 128
    for ch in range(4):
        pltpu.sync_copy(x_hbm.at[pl.ds(t0 + ch * 32, 32), :], xrow_v)
        pltpu.async_copy(xrow_v, xs_hbm.at[pos2_v.at[ch]], sem).wait()


def _ffn_body(texp_ref, xs_ref, w1_ref, b1_ref, w2_ref, b2_ref, ys_ref):
    t = pl.program_id(0)

    @pl.when(texp_ref[48 + t] == 1)
    def _():
        xv = xs_ref[...]
        w1 = w1_ref[0]
        b1 = b1_ref[0]
        w2 = w2_ref[0]
        b2 = b2_ref[0]
        h = jnp.dot(xv, w1, preferred_element_type=jnp.float32) + b1
        h = h * 0.5 * (1.0 + jax.lax.erf(h * 0.7071067811865476))
        ys_ref[...] = jnp.dot(h, w2, preferred_element_type=jnp.float32) + b2


def _combine_body(ys_hbm, pos_hbm, wn_hbm, out_hbm,
                  idx0_v, idx1_v, w0_v, w1_v, r0_v, r1_v, o_v, sem):
    cid = lax.axis_index("c")
    sid = lax.axis_index("s")
    wid = 2 * sid + cid
    tok0 = wid * 64
    pltpu.sync_copy(pos_hbm.at[pl.ds(tok0, 64)], idx0_v)
    pltpu.sync_copy(pos_hbm.at[pl.ds(_S + tok0, 64)], idx1_v)
    pltpu.sync_copy(wn_hbm.at[pl.ds(tok0, 64)], w0_v)
    pltpu.sync_copy(wn_hbm.at[pl.ds(_S + tok0, 64)], w1_v)
    for ch in range(2):
        pltpu.async_copy(ys_hbm.at[idx0_v.at[pl.ds(ch * 32, 32)]],
                         r0_v, sem).wait()
        pltpu.async_copy(ys_hbm.at[idx1_v.at[pl.ds(ch * 32, 32)]],
                         r1_v, sem).wait()
        for i in range(32):
            w0g = w0_v[pl.ds(ch * 32 + (i // 16) * 16, 16)]
            w1g = w1_v[pl.ds(ch * 32 + (i // 16) * 16, 16)]
            w0s = _splat_lane_f(w0g, i % 16)
            w1s = _splat_lane_f(w1g, i % 16)

            def kb(kk, _, i=i, w0s=w0s, w1s=w1s):
                off = pl.multiple_of(kk * 64, 64)
                for u in range(4):
                    a = r0_v[i, pl.ds(off + u * 16, 16)]
                    b = r1_v[i, pl.ds(off + u * 16, 16)]
                    o_v[i, pl.ds(off + u * 16, 16)] = a * w0s + b * w1s
                return 0

            lax.fori_loop(0, _H // 64, kb, 0)
        pltpu.sync_copy(o_v, out_hbm.at[pl.ds(tok0 + ch * 32, 32), :])


def kernel(x, Wg, W1, b1, W2, b2):
    B, S, H = x.shape
    E, _, D = W1.shape
    x2 = x.reshape(S, H)

    wg_pad = jnp.zeros((H, _LANES), jnp.float32).at[:, :E].set(Wg)

    eo, wo, aux = pl.pallas_call(
        _router_body,
        out_shape=(
            jax.ShapeDtypeStruct((S, _LANES), jnp.int32),
            jax.ShapeDtypeStruct((S, _LANES), jnp.float32),
            jax.ShapeDtypeStruct((1, 1), jnp.float32),
        ),
        in_specs=[
            pl.BlockSpec(memory_space=pltpu.VMEM),
            pl.BlockSpec(memory_space=pltpu.VMEM),
        ],
        out_specs=(
            pl.BlockSpec(memory_space=pltpu.VMEM),
            pl.BlockSpec(memory_space=pltpu.VMEM),
            pl.BlockSpec(memory_space=pltpu.SMEM),
        ),
    )(x2, wg_pad)

    eflat = jnp.concatenate([eo[:, 0], eo[:, 1]], axis=0)
    wflat = jnp.concatenate([wo[:, 0], wo[:, 1]], axis=0)

    mesh = plsc.VectorSubcoreMesh(core_axis_name="c", subcore_axis_name="s",
                                  num_cores=2, num_subcores=16)

    dispatch = pl.kernel(
        _dispatch_body,
        out_type=(
            jax.ShapeDtypeStruct((_P,), jnp.int32),
            jax.ShapeDtypeStruct((96,), jnp.int32),
            jax.ShapeDtypeStruct((_NP, H), jnp.float32),
        ),
        mesh=mesh,
        scratch_types=(
            pltpu.VMEM((256,), jnp.int32),
            pltpu.VMEM((32,), jnp.int32),
            pltpu.VMEM((512,), jnp.int32),
            pltpu.VMEM((528,), jnp.int32),
            pltpu.VMEM((16,), jnp.int32),
            pltpu.VMEM((16,), jnp.int32),
            pltpu.VMEM((4, 32), jnp.int32),
            pltpu.VMEM((96,), jnp.int32),
            pltpu.VMEM((32, H), jnp.float32),
            pltpu.VMEM_SHARED((1040,), jnp.int32),
            pltpu.SemaphoreType.DMA,
        ),
        compiler_params=pltpu.CompilerParams(needs_layout_passes=False),
    )
    pos, texp, xs = dispatch(eflat, x2)

    grid_spec = pltpu.PrefetchScalarGridSpec(
        num_scalar_prefetch=1,
        grid=(_NT,),
        in_specs=[
            pl.BlockSpec((_TS, H), lambda t, te: (t, 0)),
            pl.BlockSpec((1, H, D), lambda t, te: (te[t], 0, 0)),
            pl.BlockSpec((1, 1, D), lambda t, te: (te[t], 0, 0)),
            pl.BlockSpec((1, D, H), lambda t, te: (te[t], 0, 0)),
            pl.BlockSpec((1, 1, H), lambda t, te: (te[t], 0, 0)),
        ],
        out_specs=pl.BlockSpec((_TS, H), lambda t, te: (t, 0)),
    )
    ys = pl.pallas_call(
        _ffn_body,
        grid_spec=grid_spec,
        out_shape=jax.ShapeDtypeStruct((_NP, H), jnp.float32),
    )(texp, xs, W1, b1.reshape(E, 1, D), W2, b2.reshape(E, 1, H))

    combine = pl.kernel(
        _combine_body,
        out_type=jax.ShapeDtypeStruct((S, H), jnp.float32),
        mesh=mesh,
        scratch_types=(
            pltpu.VMEM((64,), jnp.int32),
            pltpu.VMEM((64,), jnp.int32),
            pltpu.VMEM((64,), jnp.float32),
            pltpu.VMEM((64,), jnp.float32),
            pltpu.VMEM((32, H), jnp.float32),
            pltpu.VMEM((32, H), jnp.float32),
            pltpu.VMEM((32, H), jnp.float32),
            pltpu.SemaphoreType.DMA,
        ),
        compiler_params=pltpu.CompilerParams(needs_layout_passes=False),
    )
    out = combine(ys, pos, wflat)

    return out.reshape(B, S, H), aux[0, 0]


# speedup vs baseline: 2.1384x; 1.0080x over previous
import jax
import jax.numpy as jnp
from jax import lax
from jax.experimental import pallas as pl
from jax.experimental.pallas import tpu as pltpu
from jax.experimental.pallas import tpu_sc as plsc

_LANES = 128
_S = 2048
_H = 1024
_D = 2048
_E = 8
_P = 2 * _S
_TS = 128
_NT = 40
_NP = _NT * _TS
_NSLICE = 32


def _router_body(x_ref, wg_ref, eo_ref, wo_ref, aux_ref):
    x = x_ref[...]
    wg = wg_ref[...]
    logits = jnp.dot(x, wg, preferred_element_type=jnp.float32)
    S = x.shape[0]
    lane = jax.lax.broadcasted_iota(jnp.int32, (S, _LANES), 1)
    neg = jnp.full_like(logits, -jnp.inf)
    logits = jnp.where(lane < _E, logits, neg)
    m = jnp.max(logits, axis=1, keepdims=True)
    ex = jnp.exp(logits - m)
    probs = ex / jnp.sum(ex, axis=1, keepdims=True)

    m1 = jnp.max(probs, axis=1, keepdims=True)
    big = jnp.int32(10 ** 9)
    i1 = jnp.min(jnp.where(probs == m1, lane, big), axis=1, keepdims=True)
    probs_m = jnp.where(lane == i1, -1.0, probs)
    m2 = jnp.max(probs_m, axis=1, keepdims=True)
    i2 = jnp.min(jnp.where(probs_m == m2, lane, big), axis=1, keepdims=True)

    denom = m1 + m2
    w1 = m1 / denom
    w2 = m2 / denom

    zi = jnp.zeros_like(lane)
    eo_ref[...] = jnp.where(lane == 0, i1, jnp.where(lane == 1, i2, zi))
    zf = jnp.zeros_like(probs)
    wo_ref[...] = jnp.where(lane == 0, w1, jnp.where(lane == 1, w2, zf))

    oh1 = (lane == i1).astype(jnp.float32)
    oh2 = (lane == i2).astype(jnp.float32)
    counts = jnp.sum(oh1 + oh2, axis=0, keepdims=True)
    avg_prob = jnp.mean(probs, axis=0, keepdims=True)
    aux_ref[0, 0] = jnp.float32(_E) * jnp.sum(counts * avg_prob)


def _lane16():
    return jax.lax.broadcasted_iota(jnp.int32, (16,), 0)


def _splat_last(s):
    return plsc.cummax(lax.rev(s, (0,)))


def _splat_lane_i(xvec, e):
    t = jnp.where(_lane16() == e, xvec, jnp.int32(-2147483648))
    return _splat_last(plsc.cummax(t))


def _splat_lane_f(xvec, e):
    t = jnp.where(_lane16() == e, xvec, -jnp.inf)
    return _splat_last(plsc.cummax(t))


def _count_slice(eid_v, base):
    lane = _lane16()
    cvec = jnp.zeros((16,), jnp.int32)
    for v in range(8):
        ev = eid_v[pl.ds(base + v * 16, 16)]
        for e in range(_E):
            mi = (ev == e).astype(jnp.int32)
            tot = _splat_last(jnp.cumsum(mi))
            cvec = jnp.where(lane == e, cvec + tot, cvec)
    return cvec


def _dispatch_body(eids_hbm, x_hbm, pos_hbm, texp_hbm, xs_hbm,
                   eid_v, cnt2_v, counts_v, prefix_v, beforeq_v, totincl_v,
                   pos2_v, texp_v, xrow_v, shared, sem):
    cid = lax.axis_index("c")
    sid = lax.axis_index("s")
    lane = _lane16()

    pltpu.sync_copy(eids_hbm.at[pl.ds(sid * 256, 256)], eid_v)
    for half in range(2):
        cnt2_v[pl.ds(half * 16, 16)] = _count_slice(eid_v, half * 128)
        pltpu.sync_copy(cnt2_v.at[pl.ds(half * 16, 16)],
                        shared.at[pl.ds((2 * sid + half) * 16, 16)])
    plsc.subcore_barrier()

    @pl.when(sid == 0)
    def _():
        pltpu.sync_copy(shared.at[pl.ds(0, 512)], counts_v)
        acc = jnp.zeros((16,), jnp.int32)
        for r in range(_NSLICE):
            prefix_v[pl.ds(r * 16, 16)] = acc
            acc = acc + counts_v[pl.ds(r * 16, 16)]
        prefix_v[pl.ds(512, 16)] = acc
        pltpu.sync_copy(prefix_v, shared.at[pl.ds(512, 528)])
    plsc.subcore_barrier()

    q = 2 * sid + cid
    pltpu.sync_copy(shared.at[pl.ds(512 + q * 16, 16)], beforeq_v)
    pltpu.sync_copy(shared.at[pl.ds(1024, 16)], totincl_v)
    before = beforeq_v[...]
    total = totincl_v[...]
    padded = jnp.bitwise_and(total + 127, jnp.int32(-128))
    incl = jnp.cumsum(padded)
    base = incl - padded
    start = base + before

    @pl.when(q == 0)
    def _():
        top = _splat_lane_i(incl, _E - 1)
        for j in range(3):
            tv = (lane + j * 16) * 128
            cnt = jnp.zeros((16,), jnp.int32)
            for e in range(_E):
                ie = _splat_lane_i(incl, e)
                cnt = cnt + (tv >= ie).astype(jnp.int32)
            texp_v[pl.ds(j * 16, 16)] = jnp.minimum(cnt, _E - 1)
            texp_v[pl.ds(48 + j * 16, 16)] = (tv < top).astype(jnp.int32)
        pltpu.sync_copy(texp_v, texp_hbm)

    pltpu.sync_copy(eids_hbm.at[pl.ds(q * 128, 128)],
                    eid_v.at[pl.ds(0, 128)])
    run = start
    for v in range(8):
        ev = eid_v[pl.ds(v * 16, 16)]
        posv = jnp.zeros((16,), jnp.int32)
        for e in range(_E):
            mv = ev == e
            s = jnp.cumsum(mv.astype(jnp.int32))
            re = _splat_lane_i(run, e)
            posv = jnp.where(mv, re + s - 1, posv)
            run = jnp.where(lane == e, run + _splat_last(s), run)
        pos2_v[v // 2, pl.ds((v % 2) * 16, 16)] = posv

    for ch in range(4):
        pltpu.sync_copy(pos2_v.at[ch],
                        pos_hbm.at[pl.ds(q * 128 + ch * 32, 32)])

    t0 = lax.rem(q, 16) * 128
    descs = [None] * 4
    for ch in range(4):
        buf = xrow_v.at[ch % 2]
        if ch >= 2:
            descs[ch - 2].wait()
        pltpu.sync_copy(x_hbm.at[pl.ds(t0 + ch * 32, 32), :], buf)
        descs[ch] = pltpu.async_copy(buf, xs_hbm.at[pos2_v.at[ch]], sem)
    descs[2].wait()
    descs[3].wait()


def _ffn_body(texp_ref, xs_ref, w1_ref, b1_ref, w2_ref, b2_ref, ys_ref):
    t = pl.program_id(0)

    @pl.when(texp_ref[48 + t] == 1)
    def _():
        xv = xs_ref[...]
        w1 = w1_ref[0]
        b1 = b1_ref[0]
        w2 = w2_ref[0]
        b2 = b2_ref[0]
        h = jnp.dot(xv, w1, preferred_element_type=jnp.float32) + b1
        h = h * 0.5 * (1.0 + jax.lax.erf(h * 0.7071067811865476))
        ys_ref[...] = jnp.dot(h, w2, preferred_element_type=jnp.float32) + b2


def _combine_body(ys_hbm, pos_hbm, wn_hbm, out_hbm,
                  idx0_v, idx1_v, w0_v, w1_v, r0_v, r1_v, o_v, sem):
    cid = lax.axis_index("c")
    sid = lax.axis_index("s")
    wid = 2 * sid + cid
    tok0 = wid * 64
    pltpu.sync_copy(pos_hbm.at[pl.ds(tok0, 64)], idx0_v)
    pltpu.sync_copy(pos_hbm.at[pl.ds(_S + tok0, 64)], idx1_v)
    pltpu.sync_copy(wn_hbm.at[pl.ds(tok0, 64)], w0_v)
    pltpu.sync_copy(wn_hbm.at[pl.ds(_S + tok0, 64)], w1_v)
    for ch in range(2):
        d0 = pltpu.async_copy(ys_hbm.at[idx0_v.at[pl.ds(ch * 32, 32)]],
                              r0_v, sem)
        d1 = pltpu.async_copy(ys_hbm.at[idx1_v.at[pl.ds(ch * 32, 32)]],
                              r1_v, sem)
        d0.wait()
        d1.wait()
        for i in range(32):
            w0g = w0_v[pl.ds(ch * 32 + (i // 16) * 16, 16)]
            w1g = w1_v[pl.ds(ch * 32 + (i // 16) * 16, 16)]
            w0s = _splat_lane_f(w0g, i % 16)
            w1s = _splat_lane_f(w1g, i % 16)

            def kb(kk, _, i=i, w0s=w0s, w1s=w1s):
                off = pl.multiple_of(kk * 64, 64)
                for u in range(4):
                    a = r0_v[i, pl.ds(off + u * 16, 16)]
                    b = r1_v[i, pl.ds(off + u * 16, 16)]
                    o_v[i, pl.ds(off + u * 16, 16)] = a * w0s + b * w1s
                return 0

            lax.fori_loop(0, _H // 64, kb, 0)
        pltpu.sync_copy(o_v, out_hbm.at[pl.ds(tok0 + ch * 32, 32), :])


def kernel(x, Wg, W1, b1, W2, b2):
    B, S, H = x.shape
    E, _, D = W1.shape
    x2 = x.reshape(S, H)

    wg_pad = jnp.zeros((H, _LANES), jnp.float32).at[:, :E].set(Wg)

    eo, wo, aux = pl.pallas_call(
        _router_body,
        out_shape=(
            jax.ShapeDtypeStruct((S, _LANES), jnp.int32),
            jax.ShapeDtypeStruct((S, _LANES), jnp.float32),
            jax.ShapeDtypeStruct((1, 1), jnp.float32),
        ),
        in_specs=[
            pl.BlockSpec(memory_space=pltpu.VMEM),
            pl.BlockSpec(memory_space=pltpu.VMEM),
        ],
        out_specs=(
            pl.BlockSpec(memory_space=pltpu.VMEM),
            pl.BlockSpec(memory_space=pltpu.VMEM),
            pl.BlockSpec(memory_space=pltpu.SMEM),
        ),
    )(x2, wg_pad)

    eflat = jnp.concatenate([eo[:, 0], eo[:, 1]], axis=0)
    wflat = jnp.concatenate([wo[:, 0], wo[:, 1]], axis=0)

    mesh = plsc.VectorSubcoreMesh(core_axis_name="c", subcore_axis_name="s",
                                  num_cores=2, num_subcores=16)

    dispatch = pl.kernel(
        _dispatch_body,
        out_type=(
            jax.ShapeDtypeStruct((_P,), jnp.int32),
            jax.ShapeDtypeStruct((96,), jnp.int32),
            jax.ShapeDtypeStruct((_NP, H), jnp.float32),
        ),
        mesh=mesh,
        scratch_types=(
            pltpu.VMEM((256,), jnp.int32),
            pltpu.VMEM((32,), jnp.int32),
            pltpu.VMEM((512,), jnp.int32),
            pltpu.VMEM((528,), jnp.int32),
            pltpu.VMEM((16,), jnp.int32),
            pltpu.VMEM((16,), jnp.int32),
            pltpu.VMEM((4, 32), jnp.int32),
            pltpu.VMEM((96,), jnp.int32),
            pltpu.VMEM((2, 32, H), jnp.float32),
            pltpu.VMEM_SHARED((1040,), jnp.int32),
            pltpu.SemaphoreType.DMA,
        ),
        compiler_params=pltpu.CompilerParams(needs_layout_passes=False),
    )
    pos, texp, xs = dispatch(eflat, x2)

    grid_spec = pltpu.PrefetchScalarGridSpec(
        num_scalar_prefetch=1,
        grid=(_NT,),
        in_specs=[
            pl.BlockSpec((_TS, H), lambda t, te: (t, 0)),
            pl.BlockSpec((1, H, D), lambda t, te: (te[t], 0, 0)),
            pl.BlockSpec((1, 1, D), lambda t, te: (te[t], 0, 0)),
            pl.BlockSpec((1, D, H), lambda t, te: (te[t], 0, 0)),
            pl.BlockSpec((1, 1, H), lambda t, te: (te[t], 0, 0)),
        ],
        out_specs=pl.BlockSpec((_TS, H), lambda t, te: (t, 0)),
    )
    ys = pl.pallas_call(
        _ffn_body,
        grid_spec=grid_spec,
        out_shape=jax.ShapeDtypeStruct((_NP, H), jnp.float32),
    )(texp, xs, W1, b1.reshape(E, 1, D), W2, b2.reshape(E, 1, H))

    combine = pl.kernel(
        _combine_body,
        out_type=jax.ShapeDtypeStruct((S, H), jnp.float32),
        mesh=mesh,
        scratch_types=(
            pltpu.VMEM((64,), jnp.int32),
            pltpu.VMEM((64,), jnp.int32),
            pltpu.VMEM((64,), jnp.float32),
            pltpu.VMEM((64,), jnp.float32),
            pltpu.VMEM((32, H), jnp.float32),
            pltpu.VMEM((32, H), jnp.float32),
            pltpu.VMEM((32, H), jnp.float32),
            pltpu.SemaphoreType.DMA,
        ),
        compiler_params=pltpu.CompilerParams(needs_layout_passes=False),
    )
    out = combine(ys, pos, wflat)

    return out.reshape(B, S, H), aux[0, 0]
